# baseline (speedup 1.0000x reference)
import functools

import jax
import jax.numpy as jnp
from jax import lax
from jax.experimental import pallas as pl
from jax.experimental.pallas import tpu as pltpu
from jax.experimental.pallas import tpu_sc as plsc

_B = 16384
_F = 26
_K = 32
_D = 1000000
_NW = 32
_LANES = 16

_TL = 512
_TCH = 61
_RPW = _TCH * _TL

_RW = _B // _NW
_NCH = 16
_RC = _RW // _NCH


def _tpl_issue(wid, c, Vt_hbm, colbuf, sem, n):
    r0 = wid * _RPW + c * _TL
    for kb in range(_K // 8):
        pltpu.async_copy(
            Vt_hbm.at[pl.ds(kb * 8, 8), pl.ds(r0, n)],
            colbuf.at[pl.ds(kb * 8, 8), pl.ds(0, n)], sem)


def _tpl_drain(wid, c, Vt_hbm, colbuf, sem, n):
    r0 = wid * _RPW + c * _TL
    for kb in range(_K // 8):
        pltpu.make_async_copy(
            Vt_hbm.at[pl.ds(kb * 8, 8), pl.ds(r0, n)],
            colbuf.at[pl.ds(kb * 8, 8), pl.ds(0, n)], sem).wait()


def _tps_issue(wid, c, rowbuf, out_hbm, sem, n):
    f0 = (wid * _RPW + c * _TL) * _K
    pltpu.async_copy(rowbuf.at[pl.ds(0, n * _K)],
                     out_hbm.at[pl.ds(f0, n * _K)], sem)


def _tps_drain(wid, c, rowbuf, out_hbm, sem, n):
    f0 = (wid * _RPW + c * _TL) * _K
    pltpu.make_async_copy(rowbuf.at[pl.ds(0, n * _K)],
                          out_hbm.at[pl.ds(f0, n * _K)], sem).wait()


def _tp_compute(colbuf, rowbuf, ngroups, tail_rows=0):
    iota = lax.iota(jnp.int32, _LANES)

    def g_body(g, carry):
        base = (g * _LANES + iota) * _K
        for k in range(_K):
            v = colbuf[k, pl.ds(g * _LANES, _LANES)]
            plsc.store_scatter(rowbuf, [base + k], v)
        return carry

    lax.fori_loop(0, ngroups, g_body, 0)
    if tail_rows:
        g = ngroups
        rvec = g * _LANES + iota
        base = rvec * _K
        mask = iota < tail_rows
        for k in range(_K):
            v = colbuf[k, pl.ds(g * _LANES, _LANES)]
            plsc.store_scatter(rowbuf, [base + k], v, mask=mask)


def _tp_body(Vt_hbm, out_hbm, cb0, cb1, rb0, rb1, sl0, sl1, ss0, ss1):
    info = plsc.get_sparse_core_info()
    wid = lax.axis_index("s") * info.num_cores + lax.axis_index("c")
    _tpl_issue(wid, 0, Vt_hbm, cb0, sl0, _TL)

    def body(s, carry):
        c0 = 2 * s
        c1 = 2 * s + 1
        _tpl_issue(wid, c1, Vt_hbm, cb1, sl1, _TL)
        _tpl_drain(wid, c0, Vt_hbm, cb0, sl0, _TL)

        @pl.when(s > 0)
        def _():
            _tps_drain(wid, c0, rb0, out_hbm, ss0, _TL)

        _tp_compute(cb0, rb0, _TL // _LANES)
        _tps_issue(wid, c0, rb0, out_hbm, ss0, _TL)

        @pl.when(s < _TCH // 2 - 1)
        def _():
            _tpl_issue(wid, c0 + 2, Vt_hbm, cb0, sl0, _TL)

        _tpl_drain(wid, c1, Vt_hbm, cb1, sl1, _TL)

        @pl.when(s > 0)
        def _():
            _tps_drain(wid, c1, rb1, out_hbm, ss1, _TL)

        _tp_compute(cb1, rb1, _TL // _LANES)
        _tps_issue(wid, c1, rb1, out_hbm, ss1, _TL)
        return carry

    lax.fori_loop(0, _TCH // 2, body, 0)

    c = _TCH - 1
    _tpl_issue(wid, c, Vt_hbm, cb0, sl0, _TL)
    _tpl_drain(wid, c, Vt_hbm, cb0, sl0, _TL)
    _tps_drain(wid, c, rb0, out_hbm, ss0, _TL)
    _tp_compute(cb0, rb0, _TL // _LANES)
    _tps_issue(wid, c, rb0, out_hbm, ss0, _TL)
    _tps_drain(wid, c, rb0, out_hbm, ss0, _TL)
    _tps_drain(wid, c, rb1, out_hbm, ss1, _TL)

    @pl.when(wid == _NW - 1)
    def _():
        _tpl_issue(wid, _TCH, Vt_hbm, cb1, sl1, _TL)
        _tpl_drain(wid, _TCH, Vt_hbm, cb1, sl1, _TL)
        _tp_compute(cb1, rb1, _TL // _LANES)
        _tps_issue(wid, _TCH, rb1, out_hbm, ss1, _TL)
        _tpl_issue(wid, _TCH + 1, Vt_hbm, cb0, sl0, 64)
        _tpl_drain(wid, _TCH + 1, Vt_hbm, cb0, sl0, 64)
        _tp_compute(cb0, rb0, 64 // _LANES)
        _tps_issue(wid, _TCH + 1, rb0, out_hbm, ss0, 64)
        _tps_drain(wid, _TCH, rb1, out_hbm, ss1, _TL)
        _tps_drain(wid, _TCH + 1, rb0, out_hbm, ss0, 64)


def _g_issue(c, idx_v, V_hbm, w_hbm, gbuf, wbuf, sem):
    for f in range(_F):
        idx = idx_v.at[f, pl.ds(c * _RC, _RC)]
        pltpu.async_copy(V_hbm.at[idx], gbuf.at[pl.ds(f * _RC, _RC)], sem)
        pltpu.async_copy(w_hbm.at[idx], wbuf.at[pl.ds(f * _RC, _RC)], sem)


def _g_drain(c, idx_v, V_hbm, w_hbm, gbuf, wbuf, sem):
    for f in range(_F):
        idx = idx_v.at[f, pl.ds(c * _RC, _RC)]
        pltpu.make_async_copy(
            V_hbm.at[idx], gbuf.at[pl.ds(f * _RC, _RC)], sem).wait()
        pltpu.make_async_copy(
            w_hbm.at[idx], wbuf.at[pl.ds(f * _RC, _RC)], sem).wait()


def _g_compute(c, gbuf, wbuf, outv):
    lanes = lax.iota(jnp.int32, _LANES)
    zeros = jnp.zeros((_LANES,), jnp.float32)
    for h in range(_RC // _LANES):
        wsum = zeros
        for f in range(_F):
            wsum = wsum + wbuf[pl.ds(f * _RC + h * _LANES, _LANES)]

        def row_body(i, ovec, h=h):
            acc0 = zeros
            acc1 = zeros
            q0 = zeros
            q1 = zeros
            for f in range(_F):
                row = f * _RC + h * _LANES + i
                v0 = gbuf[row, pl.ds(0, _LANES)]
                v1 = gbuf[row, pl.ds(_LANES, _LANES)]
                acc0 = acc0 + v0
                q0 = q0 + v0 * v0
                acc1 = acc1 + v1
                q1 = q1 + v1 * v1
            d = (acc0 * acc0 - q0) + (acc1 * acc1 - q1)
            t = jnp.sum(d) * jnp.float32(0.5)
            return jnp.where(lanes == i, t, ovec)

        ovec = lax.fori_loop(0, _LANES, row_body, zeros)
        outv[pl.ds(c * _RC + h * _LANES, _LANES)] = ovec + wsum


def _g_body(xt_hbm, V_hbm, w_hbm, out_hbm,
            idx_v, g0, g1, wb0, wb1, outv, sem0, sem1):
    info = plsc.get_sparse_core_info()
    wid = lax.axis_index("s") * info.num_cores + lax.axis_index("c")
    pltpu.sync_copy(xt_hbm.at[:, pl.ds(wid * _RW, _RW)], idx_v)
    _g_issue(0, idx_v, V_hbm, w_hbm, g0, wb0, sem0)

    def body(s, carry):
        c0 = 2 * s
        c1 = 2 * s + 1
        _g_issue(c1, idx_v, V_hbm, w_hbm, g1, wb1, sem1)
        _g_drain(c0, idx_v, V_hbm, w_hbm, g0, wb0, sem0)
        _g_compute(c0, g0, wb0, outv)

        @pl.when(s < _NCH // 2 - 1)
        def _():
            _g_issue(c0 + 2, idx_v, V_hbm, w_hbm, g0, wb0, sem0)

        _g_drain(c1, idx_v, V_hbm, w_hbm, g1, wb1, sem1)
        _g_compute(c1, g1, wb1, outv)
        return carry

    lax.fori_loop(0, _NCH // 2, body, 0)
    pltpu.sync_copy(outv, out_hbm.at[pl.ds(wid * _RW, _RW)])


@jax.jit
def _fm(xt, Vt, w):
    mesh = plsc.VectorSubcoreMesh(core_axis_name="c", subcore_axis_name="s")
    tp = functools.partial(
        pl.kernel,
        out_type=jax.ShapeDtypeStruct((_D * _K,), jnp.float32),
        mesh=mesh,
        compiler_params=pltpu.CompilerParams(
            needs_layout_passes=False, use_tc_tiling_on_sc=True),
        scratch_types=[
            pltpu.VMEM((_K, _TL), jnp.float32),
            pltpu.VMEM((_K, _TL), jnp.float32),
            pltpu.VMEM((_TL * _K,), jnp.float32),
            pltpu.VMEM((_TL * _K,), jnp.float32),
            pltpu.SemaphoreType.DMA,
            pltpu.SemaphoreType.DMA,
            pltpu.SemaphoreType.DMA,
            pltpu.SemaphoreType.DMA,
        ],
    )(_tp_body)
    Vlin = tp(Vt)

    g = functools.partial(
        pl.kernel,
        out_type=jax.ShapeDtypeStruct((_B,), jnp.float32),
        mesh=mesh,
        compiler_params=pltpu.CompilerParams(
            needs_layout_passes=False, use_tc_tiling_on_sc=False),
        scratch_types=[
            pltpu.VMEM((_F, _RW), jnp.int32),
            pltpu.VMEM((_F * _RC, _K), jnp.float32),
            pltpu.VMEM((_F * _RC, _K), jnp.float32),
            pltpu.VMEM((_F * _RC,), jnp.float32),
            pltpu.VMEM((_F * _RC,), jnp.float32),
            pltpu.VMEM((_RW,), jnp.float32),
            pltpu.SemaphoreType.DMA,
            pltpu.SemaphoreType.DMA,
        ],
    )(_g_body)
    return g(xt, Vlin.reshape(_D, _K), w)


def kernel(x, V, w, w0):
    return _fm(x.T, V.T, w) + w0


# speedup vs baseline: 3.8406x; 3.8406x over previous
import functools

import jax
import jax.numpy as jnp
from jax import lax
from jax.experimental import pallas as pl
from jax.experimental.pallas import tpu as pltpu
from jax.experimental.pallas import tpu_sc as plsc

_B = 16384
_F = 26
_K = 32
_D = 1000000
_NW = 32
_LANES = 16

_TL = 512
_TCH = 61
_RPW = _TCH * _TL

_RW = _B // _NW
_NCH = 16
_RC = _RW // _NCH


def _tpl_issue(wid, c, Vt_hbm, colbuf, sem, n):
    r0 = wid * _RPW + c * _TL
    for kb in range(_K // 8):
        pltpu.async_copy(
            Vt_hbm.at[pl.ds(kb * 8, 8), pl.ds(r0, n)],
            colbuf.at[pl.ds(kb * 8, 8), pl.ds(0, n)], sem)


def _tpl_drain(wid, c, Vt_hbm, colbuf, sem, n):
    r0 = wid * _RPW + c * _TL
    for kb in range(_K // 8):
        pltpu.make_async_copy(
            Vt_hbm.at[pl.ds(kb * 8, 8), pl.ds(r0, n)],
            colbuf.at[pl.ds(kb * 8, 8), pl.ds(0, n)], sem).wait()


def _tps_issue(wid, c, rowbuf, out_hbm, sem, n):
    f0 = (wid * _RPW + c * _TL) * _K
    pltpu.async_copy(rowbuf.at[pl.ds(0, n * _K)],
                     out_hbm.at[pl.ds(f0, n * _K)], sem)


def _tps_drain(wid, c, rowbuf, out_hbm, sem, n):
    f0 = (wid * _RPW + c * _TL) * _K
    pltpu.make_async_copy(rowbuf.at[pl.ds(0, n * _K)],
                          out_hbm.at[pl.ds(f0, n * _K)], sem).wait()


def _tp_compute(colbuf, rowbuf, ngroups, tail_rows=0):
    iota = lax.iota(jnp.int32, _LANES)

    @plsc.parallel_loop(0, ngroups, unroll=2)
    def g_body(g):
        rvec = g * _LANES + iota
        base = rvec * _K
        for j in range(_K):
            kvec = lax.bitwise_and(iota + j, _K - 1)
            v = plsc.load_gather(colbuf, [kvec, rvec])
            plsc.store_scatter(rowbuf, [base + kvec], v)
    if tail_rows:
        g = ngroups
        rvec = g * _LANES + iota
        base = rvec * _K
        mask = iota < tail_rows
        for j in range(_K):
            kvec = lax.bitwise_and(iota + j, _K - 1)
            v = plsc.load_gather(colbuf, [kvec, rvec], mask=mask)
            plsc.store_scatter(rowbuf, [base + kvec], v, mask=mask)


def _tp_body(Vt_hbm, out_hbm, cb0, cb1, rb0, rb1, sl0, sl1, ss0, ss1):
    info = plsc.get_sparse_core_info()
    wid = lax.axis_index("s") * info.num_cores + lax.axis_index("c")
    _tpl_issue(wid, 0, Vt_hbm, cb0, sl0, _TL)

    def body(s, carry):
        c0 = 2 * s
        c1 = 2 * s + 1
        _tpl_issue(wid, c1, Vt_hbm, cb1, sl1, _TL)
        _tpl_drain(wid, c0, Vt_hbm, cb0, sl0, _TL)

        @pl.when(s > 0)
        def _():
            _tps_drain(wid, c0, rb0, out_hbm, ss0, _TL)

        _tp_compute(cb0, rb0, _TL // _LANES)
        _tps_issue(wid, c0, rb0, out_hbm, ss0, _TL)

        @pl.when(s < _TCH // 2 - 1)
        def _():
            _tpl_issue(wid, c0 + 2, Vt_hbm, cb0, sl0, _TL)

        _tpl_drain(wid, c1, Vt_hbm, cb1, sl1, _TL)

        @pl.when(s > 0)
        def _():
            _tps_drain(wid, c1, rb1, out_hbm, ss1, _TL)

        _tp_compute(cb1, rb1, _TL // _LANES)
        _tps_issue(wid, c1, rb1, out_hbm, ss1, _TL)
        return carry

    lax.fori_loop(0, _TCH // 2, body, 0)

    c = _TCH - 1
    _tpl_issue(wid, c, Vt_hbm, cb0, sl0, _TL)
    _tpl_drain(wid, c, Vt_hbm, cb0, sl0, _TL)
    _tps_drain(wid, c, rb0, out_hbm, ss0, _TL)
    _tp_compute(cb0, rb0, _TL // _LANES)
    _tps_issue(wid, c, rb0, out_hbm, ss0, _TL)
    _tps_drain(wid, c, rb0, out_hbm, ss0, _TL)
    _tps_drain(wid, c, rb1, out_hbm, ss1, _TL)

    @pl.when(wid == _NW - 1)
    def _():
        _tpl_issue(wid, _TCH, Vt_hbm, cb1, sl1, _TL)
        _tpl_drain(wid, _TCH, Vt_hbm, cb1, sl1, _TL)
        _tp_compute(cb1, rb1, _TL // _LANES)
        _tps_issue(wid, _TCH, rb1, out_hbm, ss1, _TL)
        _tpl_issue(wid, _TCH + 1, Vt_hbm, cb0, sl0, 64)
        _tpl_drain(wid, _TCH + 1, Vt_hbm, cb0, sl0, 64)
        _tp_compute(cb0, rb0, 64 // _LANES)
        _tps_issue(wid, _TCH + 1, rb0, out_hbm, ss0, 64)
        _tps_drain(wid, _TCH, rb1, out_hbm, ss1, _TL)
        _tps_drain(wid, _TCH + 1, rb0, out_hbm, ss0, 64)


def _g_issue(c, idx_v, V_hbm, w_hbm, gbuf, wbuf, sem):
    for f in range(_F):
        idx = idx_v.at[f, pl.ds(c * _RC, _RC)]
        pltpu.async_copy(V_hbm.at[idx], gbuf.at[pl.ds(f * _RC, _RC)], sem)
        pltpu.async_copy(w_hbm.at[idx], wbuf.at[pl.ds(f * _RC, _RC)], sem)


def _g_drain(c, idx_v, V_hbm, w_hbm, gbuf, wbuf, sem):
    for f in range(_F):
        idx = idx_v.at[f, pl.ds(c * _RC, _RC)]
        pltpu.make_async_copy(
            V_hbm.at[idx], gbuf.at[pl.ds(f * _RC, _RC)], sem).wait()
        pltpu.make_async_copy(
            w_hbm.at[idx], wbuf.at[pl.ds(f * _RC, _RC)], sem).wait()


def _g_compute(c, gbuf, wbuf, outv):
    lanes = lax.iota(jnp.int32, _LANES)
    zeros = jnp.zeros((_LANES,), jnp.float32)
    for h in range(_RC // _LANES):
        wsum = zeros
        for f in range(_F):
            wsum = wsum + wbuf[pl.ds(f * _RC + h * _LANES, _LANES)]

        def row_body(i, ovec, h=h):
            acc0 = zeros
            acc1 = zeros
            q0 = zeros
            q1 = zeros
            for f in range(_F):
                row = f * _RC + h * _LANES + i
                v0 = gbuf[row, pl.ds(0, _LANES)]
                v1 = gbuf[row, pl.ds(_LANES, _LANES)]
                acc0 = acc0 + v0
                q0 = q0 + v0 * v0
                acc1 = acc1 + v1
                q1 = q1 + v1 * v1
            d = (acc0 * acc0 - q0) + (acc1 * acc1 - q1)
            t = jnp.sum(d) * jnp.float32(0.5)
            return jnp.where(lanes == i, t, ovec)

        ovec = lax.fori_loop(0, _LANES, row_body, zeros)
        outv[pl.ds(c * _RC + h * _LANES, _LANES)] = ovec + wsum


def _g_body(xt_hbm, V_hbm, w_hbm, out_hbm,
            idx_v, g0, g1, wb0, wb1, outv, sem0, sem1):
    info = plsc.get_sparse_core_info()
    wid = lax.axis_index("s") * info.num_cores + lax.axis_index("c")
    pltpu.sync_copy(xt_hbm.at[:, pl.ds(wid * _RW, _RW)], idx_v)
    _g_issue(0, idx_v, V_hbm, w_hbm, g0, wb0, sem0)

    def body(s, carry):
        c0 = 2 * s
        c1 = 2 * s + 1
        _g_issue(c1, idx_v, V_hbm, w_hbm, g1, wb1, sem1)
        _g_drain(c0, idx_v, V_hbm, w_hbm, g0, wb0, sem0)
        _g_compute(c0, g0, wb0, outv)

        @pl.when(s < _NCH // 2 - 1)
        def _():
            _g_issue(c0 + 2, idx_v, V_hbm, w_hbm, g0, wb0, sem0)

        _g_drain(c1, idx_v, V_hbm, w_hbm, g1, wb1, sem1)
        _g_compute(c1, g1, wb1, outv)
        return carry

    lax.fori_loop(0, _NCH // 2, body, 0)
    pltpu.sync_copy(outv, out_hbm.at[pl.ds(wid * _RW, _RW)])


@jax.jit
def _fm(xt, Vt, w):
    mesh = plsc.VectorSubcoreMesh(core_axis_name="c", subcore_axis_name="s")
    tp = functools.partial(
        pl.kernel,
        out_type=jax.ShapeDtypeStruct((_D * _K,), jnp.float32),
        mesh=mesh,
        compiler_params=pltpu.CompilerParams(
            needs_layout_passes=False, use_tc_tiling_on_sc=True),
        scratch_types=[
            pltpu.VMEM((_K, _TL), jnp.float32),
            pltpu.VMEM((_K, _TL), jnp.float32),
            pltpu.VMEM((_TL * _K,), jnp.float32),
            pltpu.VMEM((_TL * _K,), jnp.float32),
            pltpu.SemaphoreType.DMA,
            pltpu.SemaphoreType.DMA,
            pltpu.SemaphoreType.DMA,
            pltpu.SemaphoreType.DMA,
        ],
    )(_tp_body)
    Vlin = tp(Vt)

    g = functools.partial(
        pl.kernel,
        out_type=jax.ShapeDtypeStruct((_B,), jnp.float32),
        mesh=mesh,
        compiler_params=pltpu.CompilerParams(
            needs_layout_passes=False, use_tc_tiling_on_sc=False),
        scratch_types=[
            pltpu.VMEM((_F, _RW), jnp.int32),
            pltpu.VMEM((_F * _RC, _K), jnp.float32),
            pltpu.VMEM((_F * _RC, _K), jnp.float32),
            pltpu.VMEM((_F * _RC,), jnp.float32),
            pltpu.VMEM((_F * _RC,), jnp.float32),
            pltpu.VMEM((_RW,), jnp.float32),
            pltpu.SemaphoreType.DMA,
            pltpu.SemaphoreType.DMA,
        ],
    )(_g_body)
    return g(xt, Vlin.reshape(_D, _K), w)


def kernel(x, V, w, w0):
    return _fm(x.T, V.T, w) + w0


# speedup vs baseline: 3.9621x; 1.0316x over previous
import functools

import jax
import jax.numpy as jnp
from jax import lax
from jax.experimental import pallas as pl
from jax.experimental.pallas import tpu as pltpu
from jax.experimental.pallas import tpu_sc as plsc

_B = 16384
_F = 26
_K = 32
_D = 1000000
_NW = 32
_LANES = 16

_TL = 512
_TCH = 61
_RPW = _TCH * _TL

_RW = _B // _NW
_NCH = 16
_RC = _RW // _NCH


def _tpl_issue(wid, c, Vt_hbm, colbuf, sem, n):
    r0 = wid * _RPW + c * _TL
    for kb in range(_K // 8):
        pltpu.async_copy(
            Vt_hbm.at[pl.ds(kb * 8, 8), pl.ds(r0, n)],
            colbuf.at[pl.ds(kb * 8, 8), pl.ds(0, n)], sem)


def _tpl_drain(wid, c, Vt_hbm, colbuf, sem, n):
    r0 = wid * _RPW + c * _TL
    for kb in range(_K // 8):
        pltpu.make_async_copy(
            Vt_hbm.at[pl.ds(kb * 8, 8), pl.ds(r0, n)],
            colbuf.at[pl.ds(kb * 8, 8), pl.ds(0, n)], sem).wait()


def _tps_issue(wid, c, rowbuf, out_hbm, sem, n):
    f0 = (wid * _RPW + c * _TL) * _K
    pltpu.async_copy(rowbuf.at[pl.ds(0, n * _K)],
                     out_hbm.at[pl.ds(f0, n * _K)], sem)


def _tps_drain(wid, c, rowbuf, out_hbm, sem, n):
    f0 = (wid * _RPW + c * _TL) * _K
    pltpu.make_async_copy(rowbuf.at[pl.ds(0, n * _K)],
                          out_hbm.at[pl.ds(f0, n * _K)], sem).wait()


def _tp_compute(colbuf, rowbuf, ngroups, tail_rows=0):
    iota = lax.iota(jnp.int32, _LANES)

    @plsc.parallel_loop(0, ngroups, unroll=4)
    def g_body(g):
        rvec = g * _LANES + iota
        base = rvec * _K
        for j in range(_K):
            kvec = lax.bitwise_and(iota + j, _K - 1)
            v = plsc.load_gather(colbuf, [kvec, rvec])
            plsc.store_scatter(rowbuf, [base + kvec], v)
    if tail_rows:
        g = ngroups
        rvec = g * _LANES + iota
        base = rvec * _K
        mask = iota < tail_rows
        for j in range(_K):
            kvec = lax.bitwise_and(iota + j, _K - 1)
            v = plsc.load_gather(colbuf, [kvec, rvec], mask=mask)
            plsc.store_scatter(rowbuf, [base + kvec], v, mask=mask)


def _tp_body(Vt_hbm, out_hbm, cb0, cb1, rb0, rb1, sl0, sl1, ss0, ss1):
    info = plsc.get_sparse_core_info()
    wid = lax.axis_index("s") * info.num_cores + lax.axis_index("c")
    _tpl_issue(wid, 0, Vt_hbm, cb0, sl0, _TL)

    def body(s, carry):
        c0 = 2 * s
        c1 = 2 * s + 1
        _tpl_issue(wid, c1, Vt_hbm, cb1, sl1, _TL)
        _tpl_drain(wid, c0, Vt_hbm, cb0, sl0, _TL)

        @pl.when(s > 0)
        def _():
            _tps_drain(wid, c0, rb0, out_hbm, ss0, _TL)

        _tp_compute(cb0, rb0, _TL // _LANES)
        _tps_issue(wid, c0, rb0, out_hbm, ss0, _TL)

        @pl.when(s < _TCH // 2 - 1)
        def _():
            _tpl_issue(wid, c0 + 2, Vt_hbm, cb0, sl0, _TL)

        _tpl_drain(wid, c1, Vt_hbm, cb1, sl1, _TL)

        @pl.when(s > 0)
        def _():
            _tps_drain(wid, c1, rb1, out_hbm, ss1, _TL)

        _tp_compute(cb1, rb1, _TL // _LANES)
        _tps_issue(wid, c1, rb1, out_hbm, ss1, _TL)
        return carry

    lax.fori_loop(0, _TCH // 2, body, 0)

    c = _TCH - 1
    _tpl_issue(wid, c, Vt_hbm, cb0, sl0, _TL)
    _tpl_drain(wid, c, Vt_hbm, cb0, sl0, _TL)
    _tps_drain(wid, c, rb0, out_hbm, ss0, _TL)
    _tp_compute(cb0, rb0, _TL // _LANES)
    _tps_issue(wid, c, rb0, out_hbm, ss0, _TL)
    _tps_drain(wid, c, rb0, out_hbm, ss0, _TL)
    _tps_drain(wid, c, rb1, out_hbm, ss1, _TL)

    @pl.when(wid == _NW - 1)
    def _():
        _tpl_issue(wid, _TCH, Vt_hbm, cb1, sl1, _TL)
        _tpl_drain(wid, _TCH, Vt_hbm, cb1, sl1, _TL)
        _tp_compute(cb1, rb1, _TL // _LANES)
        _tps_issue(wid, _TCH, rb1, out_hbm, ss1, _TL)
        _tpl_issue(wid, _TCH + 1, Vt_hbm, cb0, sl0, 64)
        _tpl_drain(wid, _TCH + 1, Vt_hbm, cb0, sl0, 64)
        _tp_compute(cb0, rb0, 64 // _LANES)
        _tps_issue(wid, _TCH + 1, rb0, out_hbm, ss0, 64)
        _tps_drain(wid, _TCH, rb1, out_hbm, ss1, _TL)
        _tps_drain(wid, _TCH + 1, rb0, out_hbm, ss0, 64)


def _g_issue(c, idx_v, V_hbm, w_hbm, gbuf, wbuf, sem):
    for f in range(_F):
        idx = idx_v.at[f, pl.ds(c * _RC, _RC)]
        pltpu.async_copy(V_hbm.at[idx], gbuf.at[pl.ds(f * _RC, _RC)], sem)
        pltpu.async_copy(w_hbm.at[idx], wbuf.at[pl.ds(f * _RC, _RC)], sem)


def _g_drain(c, idx_v, V_hbm, w_hbm, gbuf, wbuf, sem):
    for f in range(_F):
        idx = idx_v.at[f, pl.ds(c * _RC, _RC)]
        pltpu.make_async_copy(
            V_hbm.at[idx], gbuf.at[pl.ds(f * _RC, _RC)], sem).wait()
        pltpu.make_async_copy(
            w_hbm.at[idx], wbuf.at[pl.ds(f * _RC, _RC)], sem).wait()


def _g_compute(c, gbuf, wbuf, outv):
    lanes = lax.iota(jnp.int32, _LANES)
    zeros = jnp.zeros((_LANES,), jnp.float32)
    for h in range(_RC // _LANES):
        wsum = zeros
        for f in range(_F):
            wsum = wsum + wbuf[pl.ds(f * _RC + h * _LANES, _LANES)]

        def row_body(i, ovec, h=h):
            acc0 = zeros
            acc1 = zeros
            q0 = zeros
            q1 = zeros
            for f in range(_F):
                row = f * _RC + h * _LANES + i
                v0 = gbuf[row, pl.ds(0, _LANES)]
                v1 = gbuf[row, pl.ds(_LANES, _LANES)]
                acc0 = acc0 + v0
                q0 = q0 + v0 * v0
                acc1 = acc1 + v1
                q1 = q1 + v1 * v1
            d = (acc0 * acc0 - q0) + (acc1 * acc1 - q1)
            t = jnp.sum(d) * jnp.float32(0.5)
            return jnp.where(lanes == i, t, ovec)

        ovec = plsc.parallel_loop(0, _LANES, carry=zeros)(row_body)
        outv[pl.ds(c * _RC + h * _LANES, _LANES)] = ovec + wsum


def _g_body(xt_hbm, V_hbm, w_hbm, out_hbm,
            idx_v, g0, g1, wb0, wb1, outv, sem0, sem1):
    info = plsc.get_sparse_core_info()
    wid = lax.axis_index("s") * info.num_cores + lax.axis_index("c")
    pltpu.sync_copy(xt_hbm.at[:, pl.ds(wid * _RW, _RW)], idx_v)
    _g_issue(0, idx_v, V_hbm, w_hbm, g0, wb0, sem0)

    def body(s, carry):
        c0 = 2 * s
        c1 = 2 * s + 1
        _g_issue(c1, idx_v, V_hbm, w_hbm, g1, wb1, sem1)
        _g_drain(c0, idx_v, V_hbm, w_hbm, g0, wb0, sem0)
        _g_compute(c0, g0, wb0, outv)

        @pl.when(s < _NCH // 2 - 1)
        def _():
            _g_issue(c0 + 2, idx_v, V_hbm, w_hbm, g0, wb0, sem0)

        _g_drain(c1, idx_v, V_hbm, w_hbm, g1, wb1, sem1)
        _g_compute(c1, g1, wb1, outv)
        return carry

    lax.fori_loop(0, _NCH // 2, body, 0)
    pltpu.sync_copy(outv, out_hbm.at[pl.ds(wid * _RW, _RW)])


@jax.jit
def _fm(xt, Vt, w):
    mesh = plsc.VectorSubcoreMesh(core_axis_name="c", subcore_axis_name="s")
    tp = functools.partial(
        pl.kernel,
        out_type=jax.ShapeDtypeStruct((_D * _K,), jnp.float32),
        mesh=mesh,
        compiler_params=pltpu.CompilerParams(
            needs_layout_passes=False, use_tc_tiling_on_sc=True),
        scratch_types=[
            pltpu.VMEM((_K, _TL), jnp.float32),
            pltpu.VMEM((_K, _TL), jnp.float32),
            pltpu.VMEM((_TL * _K,), jnp.float32),
            pltpu.VMEM((_TL * _K,), jnp.float32),
            pltpu.SemaphoreType.DMA,
            pltpu.SemaphoreType.DMA,
            pltpu.SemaphoreType.DMA,
            pltpu.SemaphoreType.DMA,
        ],
    )(_tp_body)
    Vlin = tp(Vt)

    g = functools.partial(
        pl.kernel,
        out_type=jax.ShapeDtypeStruct((_B,), jnp.float32),
        mesh=mesh,
        compiler_params=pltpu.CompilerParams(
            needs_layout_passes=False, use_tc_tiling_on_sc=False),
        scratch_types=[
            pltpu.VMEM((_F, _RW), jnp.int32),
            pltpu.VMEM((_F * _RC, _K), jnp.float32),
            pltpu.VMEM((_F * _RC, _K), jnp.float32),
            pltpu.VMEM((_F * _RC,), jnp.float32),
            pltpu.VMEM((_F * _RC,), jnp.float32),
            pltpu.VMEM((_RW,), jnp.float32),
            pltpu.SemaphoreType.DMA,
            pltpu.SemaphoreType.DMA,
        ],
    )(_g_body)
    return g(xt, Vlin.reshape(_D, _K), w)


def kernel(x, V, w, w0):
    return _fm(x.T, V.T, w) + w0
